# baseline (speedup 1.0000x reference)
import functools

import numpy as np
import jax
import jax.numpy as jnp
from jax import lax
from jax.experimental import pallas as pl
from jax.experimental.pallas import tpu as pltpu
from jax.experimental.pallas import tpu_sc as plsc

_SEED = 1024
_HWS = 4194304
_BATCH = 16384
_DIM = 64
_r = np.random.RandomState(_SEED)
_rn = np.concatenate([np.array([2038074743]), _r.randint(0, 2038074743, (10,))]).astype(np.int64)
_P, _A, _B, _R = int(_rn[0]), int(_rn[1]), int(_rn[2]), int(_rn[3])
_S = (_R << 10) % _P
_THI = ((np.arange(1024, dtype=np.int64) * _S) % _P).astype(np.int32)
_TLO = (((np.arange(1024, dtype=np.int64) * _R) % _P) - _P).astype(np.int32)
_CPR = (((np.arange(_DIM, dtype=np.int64) * _B + _A) % _P) - _P).astype(np.int32)
_CPR_LIST = [int(x) for x in _CPR]
_MASK = _HWS - 1

_NW = 32
_IPW = _BATCH // _NW
_EPW = _IPW * _DIM
_ROWS = _EPW // 128
_NG = _IPW // 16
_RPG = 16 * _DIM // 128


def _make_sc_kernel():
    mesh = plsc.VectorSubcoreMesh(core_axis_name="c", subcore_axis_name="s")

    @functools.partial(
        pl.kernel,
        mesh=mesh,
        compiler_params=pltpu.CompilerParams(needs_layout_passes=False),
        out_type=jax.ShapeDtypeStruct((_BATCH * _DIM,), jnp.float32),
        scratch_types=[
            pltpu.VMEM((_IPW,), jnp.int32),
            pltpu.VMEM((1024,), jnp.int32),
            pltpu.VMEM((1024,), jnp.int32),
            pltpu.VMEM((_ROWS, 128), jnp.int32),
            pltpu.VMEM((_EPW,), jnp.float32),
            pltpu.SemaphoreType.DMA,
        ],
    )
    def robez(idx_hbm, hw_hbm, thi_hbm, tlo_hbm, out_hbm,
              idx_v, thi_v, tlo_v, hidx_v, vals_v, sem):
        wid = (lax.axis_index("s") * jnp.int32(2)
               + lax.axis_index("c")).astype(jnp.int32)
        base = wid * jnp.int32(_IPW)

        pltpu.sync_copy(idx_hbm.at[pl.ds(base, _IPW)], idx_v)
        pltpu.sync_copy(thi_hbm, thi_v)
        pltpu.sync_copy(tlo_hbm, tlo_v)

        c10 = jnp.int32(10)
        c31 = jnp.int32(31)
        c1023 = jnp.int32(1023)
        cP = jnp.int32(_P)
        cMask = jnp.int32(_MASK)
        lane = lax.iota(jnp.int32, 16)
        row0 = lax.shift_right_logical(lane, jnp.int32(1))
        col0 = lax.bitwise_and(lane, jnp.int32(1)) * jnp.int32(64)

        def gbody(g, _):
            iv = idx_v[pl.ds(g * jnp.int32(16), 16)]
            a = lax.shift_right_logical(iv, c10)
            b = lax.bitwise_and(iv, c1023)
            v = plsc.load_gather(thi_v, [a]) + plsc.load_gather(tlo_v, [b])
            f = v + lax.bitwise_and(cP, lax.shift_right_arithmetic(v, c31))
            rowg = row0 + g * jnp.int32(_RPG)
            for d in range(_DIM):
                w = f + jnp.int32(_CPR_LIST[d])
                m = w + lax.bitwise_and(cP, lax.shift_right_arithmetic(w, c31))
                h = lax.bitwise_and(m, cMask)
                plsc.store_scatter(hidx_v, [rowg, col0 + jnp.int32(d)], h)
            for k in range(_RPG):
                j = g * jnp.int32(_RPG) + jnp.int32(k)
                pltpu.make_async_copy(
                    hw_hbm.at[hidx_v.at[j]],
                    vals_v.at[pl.ds(j * jnp.int32(128), 128)],
                    sem).start()
            return _
        lax.fori_loop(jnp.int32(0), jnp.int32(_NG), gbody, None)

        pltpu.make_async_copy(out_hbm.at[pl.ds(0, _EPW)], vals_v, sem).wait()

        pltpu.sync_copy(vals_v, out_hbm.at[pl.ds(wid * jnp.int32(_EPW), _EPW)])

    return robez


_sc_kernel = _make_sc_kernel()


def kernel(indices, hashed_weights):
    idx32 = indices.astype(jnp.int32)
    out = _sc_kernel(idx32, hashed_weights,
                     jnp.asarray(_THI), jnp.asarray(_TLO))
    return out.reshape(_BATCH, _DIM)


# speedup vs baseline: 1.0164x; 1.0164x over previous
import functools

import numpy as np
import jax
import jax.numpy as jnp
from jax import lax
from jax.experimental import pallas as pl
from jax.experimental.pallas import tpu as pltpu
from jax.experimental.pallas import tpu_sc as plsc

_SEED = 1024
_HWS = 4194304
_BATCH = 16384
_DIM = 64
_r = np.random.RandomState(_SEED)
_rn = np.concatenate([np.array([2038074743]), _r.randint(0, 2038074743, (10,))]).astype(np.int64)
_P, _A, _B, _R = int(_rn[0]), int(_rn[1]), int(_rn[2]), int(_rn[3])
_S = (_R << 10) % _P
_THI = ((np.arange(1024, dtype=np.int64) * _S) % _P).astype(np.int32)
_TLO = (((np.arange(1024, dtype=np.int64) * _R) % _P) - _P).astype(np.int32)
_CPR = (((np.arange(_DIM, dtype=np.int64) * _B + _A) % _P) - _P).astype(np.int32)
_CPR_LIST = [int(x) for x in _CPR]
_MASK = _HWS - 1

_NW = 32
_IPW = _BATCH // _NW
_EPW = _IPW * _DIM
_ROWS = _EPW // 128
_NG = _IPW // 16
_RPG = 16 * _DIM // 128


def _make_sc_kernel():
    mesh = plsc.VectorSubcoreMesh(core_axis_name="c", subcore_axis_name="s")

    @functools.partial(
        pl.kernel,
        mesh=mesh,
        compiler_params=pltpu.CompilerParams(needs_layout_passes=False),
        out_type=jax.ShapeDtypeStruct((_BATCH * _DIM,), jnp.float32),
        scratch_types=[
            pltpu.VMEM((_IPW,), jnp.int32),
            pltpu.VMEM((1024,), jnp.int32),
            pltpu.VMEM((1024,), jnp.int32),
            pltpu.VMEM((_ROWS, 128), jnp.int32),
            pltpu.VMEM((_EPW,), jnp.float32),
            pltpu.SemaphoreType.DMA,
            pltpu.SemaphoreType.DMA,
        ],
    )
    def robez(idx_hbm, hw_hbm, thi_hbm, tlo_hbm, out_hbm,
              idx_v, thi_v, tlo_v, hidx_v, vals_v, sem, sem_in):
        wid = (lax.axis_index("s") * jnp.int32(2)
               + lax.axis_index("c")).astype(jnp.int32)
        base = wid * jnp.int32(_IPW)

        cp_idx = pltpu.make_async_copy(
            idx_hbm.at[pl.ds(base, _IPW)], idx_v, sem_in)
        cp_thi = pltpu.make_async_copy(thi_hbm, thi_v, sem_in)
        cp_tlo = pltpu.make_async_copy(tlo_hbm, tlo_v, sem_in)
        cp_idx.start()
        cp_thi.start()
        cp_tlo.start()
        cp_idx.wait()
        cp_thi.wait()
        cp_tlo.wait()

        c10 = jnp.int32(10)
        c31 = jnp.int32(31)
        c1023 = jnp.int32(1023)
        cP = jnp.int32(_P)
        cMask = jnp.int32(_MASK)
        lane = lax.iota(jnp.int32, 16)
        row0 = lax.shift_right_logical(lane, jnp.int32(1))
        col0 = lax.bitwise_and(lane, jnp.int32(1)) * jnp.int32(64)

        def gbody(g, _):
            iv = idx_v[pl.ds(g * jnp.int32(16), 16)]
            a = lax.shift_right_logical(iv, c10)
            b = lax.bitwise_and(iv, c1023)
            v = plsc.load_gather(thi_v, [a]) + plsc.load_gather(tlo_v, [b])
            f = v + lax.bitwise_and(cP, lax.shift_right_arithmetic(v, c31))
            rowg = row0 + g * jnp.int32(_RPG)
            for d in range(_DIM):
                w = f + jnp.int32(_CPR_LIST[d])
                m = w + lax.bitwise_and(cP, lax.shift_right_arithmetic(w, c31))
                h = lax.bitwise_and(m, cMask)
                plsc.store_scatter(hidx_v, [rowg, col0 + jnp.int32(d)], h)
            for k in range(_RPG):
                j = g * jnp.int32(_RPG) + jnp.int32(k)
                pltpu.make_async_copy(
                    hw_hbm.at[hidx_v.at[j]],
                    vals_v.at[pl.ds(j * jnp.int32(128), 128)],
                    sem).start()
            return _
        lax.fori_loop(jnp.int32(0), jnp.int32(_NG), gbody, None)

        pltpu.make_async_copy(out_hbm.at[pl.ds(0, _EPW)], vals_v, sem).wait()

        pltpu.sync_copy(vals_v, out_hbm.at[pl.ds(wid * jnp.int32(_EPW), _EPW)])

    return robez


_sc_kernel = _make_sc_kernel()


def kernel(indices, hashed_weights):
    idx32 = indices.astype(jnp.int32)
    out = _sc_kernel(idx32, hashed_weights,
                     jnp.asarray(_THI), jnp.asarray(_TLO))
    return out.reshape(_BATCH, _DIM)


# speedup vs baseline: 1.0170x; 1.0005x over previous
import functools

import numpy as np
import jax
import jax.numpy as jnp
from jax import lax
from jax.experimental import pallas as pl
from jax.experimental.pallas import tpu as pltpu
from jax.experimental.pallas import tpu_sc as plsc

_SEED = 1024
_HWS = 4194304
_BATCH = 16384
_DIM = 64
_r = np.random.RandomState(_SEED)
_rn = np.concatenate([np.array([2038074743]), _r.randint(0, 2038074743, (10,))]).astype(np.int64)
_P, _A, _B, _R = int(_rn[0]), int(_rn[1]), int(_rn[2]), int(_rn[3])
_S = (_R << 10) % _P
_THI = ((np.arange(1024, dtype=np.int64) * _S) % _P).astype(np.int32)
_TLO = (((np.arange(1024, dtype=np.int64) * _R) % _P) - _P).astype(np.int32)
_CPR = (((np.arange(_DIM, dtype=np.int64) * _B + _A) % _P) - _P).astype(np.int32)
_CPR_LIST = [int(x) for x in _CPR]
_MASK = _HWS - 1

_NW = 32
_IPW = _BATCH // _NW
_EPW = _IPW * _DIM
_ROWS = _EPW // 128
_NG = _IPW // 16
_RPG = 16 * _DIM // 128


def _make_sc_kernel():
    mesh = plsc.VectorSubcoreMesh(core_axis_name="c", subcore_axis_name="s")

    @functools.partial(
        pl.kernel,
        mesh=mesh,
        compiler_params=pltpu.CompilerParams(needs_layout_passes=False),
        out_type=jax.ShapeDtypeStruct((_BATCH * _DIM,), jnp.float32),
        scratch_types=[
            pltpu.VMEM((_IPW,), jnp.int32),
            pltpu.VMEM((1024,), jnp.int32),
            pltpu.VMEM((1024,), jnp.int32),
            pltpu.VMEM((_ROWS, 128), jnp.int32),
            pltpu.VMEM((_EPW,), jnp.float32),
            pltpu.SemaphoreType.DMA,
            pltpu.SemaphoreType.DMA,
        ],
    )
    def robez(idx_hbm, hw_hbm, thi_hbm, tlo_hbm, out_hbm,
              idx_v, thi_v, tlo_v, hidx_v, vals_v, sem, sem_in):
        wid = (lax.axis_index("s") * jnp.int32(2)
               + lax.axis_index("c")).astype(jnp.int32)
        base = wid * jnp.int32(_IPW)

        cp_idx = pltpu.make_async_copy(
            idx_hbm.at[pl.ds(base, _IPW)], idx_v, sem_in)
        cp_thi = pltpu.make_async_copy(thi_hbm, thi_v, sem_in)
        cp_tlo = pltpu.make_async_copy(tlo_hbm, tlo_v, sem_in)
        cp_idx.start()
        cp_thi.start()
        cp_tlo.start()
        cp_idx.wait()
        cp_thi.wait()
        cp_tlo.wait()

        c10 = jnp.int32(10)
        c31 = jnp.int32(31)
        c1023 = jnp.int32(1023)
        cP = jnp.int32(_P)
        cMask = jnp.int32(_MASK)
        lane = lax.iota(jnp.int32, 16)
        row0 = lax.shift_right_logical(lane, jnp.int32(1))
        col0 = lax.bitwise_and(lane, jnp.int32(1)) * jnp.int32(64)

        def gbody(g, _):
            iv = idx_v[pl.ds(g * jnp.int32(16), 16)]
            a = lax.shift_right_logical(iv, c10)
            b = lax.bitwise_and(iv, c1023)
            v = plsc.load_gather(thi_v, [a]) + plsc.load_gather(tlo_v, [b])
            f = v + lax.bitwise_and(cP, lax.shift_right_arithmetic(v, c31))
            rowg = row0 + g * jnp.int32(_RPG)
            for d in range(_DIM):
                w = f + jnp.int32(_CPR_LIST[d])
                m = w + lax.bitwise_and(cP, lax.shift_right_arithmetic(w, c31))
                h = lax.bitwise_and(m, cMask)
                plsc.store_scatter(hidx_v, [rowg, col0 + jnp.int32(d)], h)
            for k in range(_RPG):
                j = g * jnp.int32(_RPG) + jnp.int32(k)
                pltpu.make_async_copy(
                    hw_hbm.at[hidx_v.at[j]],
                    vals_v.at[pl.ds(j * jnp.int32(128), 128)],
                    sem if k % 2 == 0 else sem_in).start()
            return _
        lax.fori_loop(jnp.int32(0), jnp.int32(_NG), gbody, None)

        half = _EPW // 2
        pltpu.make_async_copy(
            out_hbm.at[pl.ds(0, half)], vals_v.at[pl.ds(0, half)], sem).wait()
        pltpu.make_async_copy(
            out_hbm.at[pl.ds(0, half)],
            vals_v.at[pl.ds(half, half)], sem_in).wait()

        pltpu.sync_copy(vals_v, out_hbm.at[pl.ds(wid * jnp.int32(_EPW), _EPW)])

    return robez


_sc_kernel = _make_sc_kernel()


def kernel(indices, hashed_weights):
    idx32 = indices.astype(jnp.int32)
    out = _sc_kernel(idx32, hashed_weights,
                     jnp.asarray(_THI), jnp.asarray(_TLO))
    return out.reshape(_BATCH, _DIM)


# speedup vs baseline: 1.1043x; 1.0858x over previous
import functools

import numpy as np
import jax
import jax.numpy as jnp
from jax import lax
from jax.experimental import pallas as pl
from jax.experimental.pallas import tpu as pltpu
from jax.experimental.pallas import tpu_sc as plsc

_SEED = 1024
_HWS = 4194304
_BATCH = 16384
_DIM = 64
_r = np.random.RandomState(_SEED)
_rn = np.concatenate([np.array([2038074743]), _r.randint(0, 2038074743, (10,))]).astype(np.int64)
_P, _A, _B, _R = int(_rn[0]), int(_rn[1]), int(_rn[2]), int(_rn[3])
_S = (_R << 10) % _P
_THI = ((np.arange(1024, dtype=np.int64) * _S) % _P).astype(np.int32)
_TLO = (((np.arange(1024, dtype=np.int64) * _R) % _P) - _P).astype(np.int32)
_CPR = (((np.arange(_DIM, dtype=np.int64) * _B + _A) % _P) - _P).astype(np.int32)
_CPR_LIST = [int(x) for x in _CPR]
_MASK = _HWS - 1

_NW = 32
_IPW = _BATCH // _NW
_EPW = _IPW * _DIM
_ROWS = _EPW // 128
_NG = _IPW // 16
_RPG = 16 * _DIM // 128


def _make_sc_kernel():
    mesh = plsc.VectorSubcoreMesh(core_axis_name="c", subcore_axis_name="s")

    @functools.partial(
        pl.kernel,
        mesh=mesh,
        compiler_params=pltpu.CompilerParams(needs_layout_passes=False),
        out_type=jax.ShapeDtypeStruct((_BATCH, 128), jnp.float32),
        scratch_types=[
            pltpu.VMEM((_IPW,), jnp.int32),
            pltpu.VMEM((1024,), jnp.int32),
            pltpu.VMEM((1024,), jnp.int32),
            pltpu.VMEM((_ROWS, 128), jnp.int32),
            pltpu.VMEM((_IPW, 128), jnp.float32),
            pltpu.SemaphoreType.DMA,
            pltpu.SemaphoreType.DMA,
        ],
    )
    def robez(idx_hbm, hw_hbm, thi_hbm, tlo_hbm, out_hbm,
              idx_v, thi_v, tlo_v, hidx_v, vals_v, sem, sem_in):
        wid = (lax.axis_index("s") * jnp.int32(2)
               + lax.axis_index("c")).astype(jnp.int32)
        base = wid * jnp.int32(_IPW)

        cp_idx = pltpu.make_async_copy(
            idx_hbm.at[pl.ds(base, _IPW)], idx_v, sem_in)
        cp_thi = pltpu.make_async_copy(thi_hbm, thi_v, sem_in)
        cp_tlo = pltpu.make_async_copy(tlo_hbm, tlo_v, sem_in)
        cp_idx.start()
        cp_thi.start()
        cp_tlo.start()
        cp_idx.wait()
        cp_thi.wait()
        cp_tlo.wait()

        c10 = jnp.int32(10)
        c31 = jnp.int32(31)
        c1023 = jnp.int32(1023)
        cP = jnp.int32(_P)
        cMask = jnp.int32(_MASK)
        lane = lax.iota(jnp.int32, 16)
        row0 = lax.shift_right_logical(lane, jnp.int32(1))
        col0 = lax.bitwise_and(lane, jnp.int32(1)) * jnp.int32(64)

        def gbody(g, _):
            iv = idx_v[pl.ds(g * jnp.int32(16), 16)]
            a = lax.shift_right_logical(iv, c10)
            b = lax.bitwise_and(iv, c1023)
            v = plsc.load_gather(thi_v, [a]) + plsc.load_gather(tlo_v, [b])
            f = v + lax.bitwise_and(cP, lax.shift_right_arithmetic(v, c31))
            rowg = row0 + g * jnp.int32(_RPG)
            for d in range(_DIM):
                w = f + jnp.int32(_CPR_LIST[d])
                m = w + lax.bitwise_and(cP, lax.shift_right_arithmetic(w, c31))
                h = lax.bitwise_and(m, cMask)
                plsc.store_scatter(hidx_v, [rowg, col0 + jnp.int32(d)], h)
            g8 = g * jnp.int32(_RPG)
            g16 = g * jnp.int32(16)
            for k in range(16):
                row = g8 + jnp.int32(k >> 1)
                cb = (k & 1) * 64
                pltpu.make_async_copy(
                    hw_hbm.at[hidx_v.at[row, pl.ds(cb, 64)]],
                    vals_v.at[g16 + jnp.int32(k), pl.ds(0, 64)],
                    sem).start()
            return _
        lax.fori_loop(jnp.int32(0), jnp.int32(_NG), gbody, None)

        pltpu.make_async_copy(
            out_hbm.at[pl.ds(0, _ROWS)], vals_v.at[pl.ds(0, _ROWS)], sem).wait()

        pltpu.sync_copy(vals_v, out_hbm.at[pl.ds(base, _IPW)])

    return robez


_sc_kernel = _make_sc_kernel()


def kernel(indices, hashed_weights):
    idx32 = indices.astype(jnp.int32)
    out = _sc_kernel(idx32, hashed_weights,
                     jnp.asarray(_THI), jnp.asarray(_TLO))
    return out[:, :_DIM]
